# baseline (speedup 1.0000x reference)
import functools

import jax
import jax.numpy as jnp
from jax import lax
from jax.experimental import pallas as pl
from jax.experimental.pallas import tpu as pltpu
from jax.experimental.pallas import tpu_sc as plsc

_B = 16384
_D = 128
_NC = 2
_NS = 16
_NW = _NC * _NS
_BPW = _B // _NW


def _body(h_hbm, dom_hbm, emb_hbm, out_hbm, idx_v, hbuf, sem_i, sem_h, sem_e,
          sem_e2, sem_o, sem_o2):
    wid = lax.axis_index("s") * _NC + lax.axis_index("c")
    base = wid * _BPW
    rows = pl.ds(base, _BPW)

    idx_cp = pltpu.make_async_copy(dom_hbm.at[rows], idx_v, sem_i)
    idx_cp.start()
    h_cp = pltpu.make_async_copy(h_hbm.at[rows], hbuf, sem_h)
    h_cp.start()
    idx_cp.wait()
    h_cp.wait()

    half = _BPW // 2
    g0 = pltpu.async_copy(
        emb_hbm.at[idx_v.at[pl.ds(0, half)]],
        hbuf.at[pl.ds(0, half)], sem_e, add=True)
    g1 = pltpu.async_copy(
        emb_hbm.at[idx_v.at[pl.ds(half, half)]],
        hbuf.at[pl.ds(half, half)], sem_e2, add=True)
    g0.wait()
    w0 = pltpu.make_async_copy(
        hbuf.at[pl.ds(0, half)], out_hbm.at[pl.ds(base, half)], sem_o)
    w0.start()
    g1.wait()
    w1 = pltpu.make_async_copy(
        hbuf.at[pl.ds(half, half)], out_hbm.at[pl.ds(base + half, half)],
        sem_o2)
    w1.start()
    w0.wait()
    w1.wait()


@jax.jit
def _domain_token(h, domain, emb):
    mesh = plsc.VectorSubcoreMesh(core_axis_name="c", subcore_axis_name="s")
    return pl.kernel(
        _body,
        out_type=jax.ShapeDtypeStruct((_B, _D), jnp.float32),
        mesh=mesh,
        scratch_types=[
            pltpu.VMEM((_BPW,), jnp.int32),
            pltpu.VMEM((_BPW, _D), jnp.float32),
            pltpu.SemaphoreType.DMA,
            pltpu.SemaphoreType.DMA,
            pltpu.SemaphoreType.DMA,
            pltpu.SemaphoreType.DMA,
            pltpu.SemaphoreType.DMA,
            pltpu.SemaphoreType.DMA,
        ],
    )(h, domain, emb)


def kernel(h, domain, emb):
    return _domain_token(h, domain.astype(jnp.int32), emb)


# speedup vs baseline: 1.0277x; 1.0277x over previous
import functools

import jax
import jax.numpy as jnp
from jax import lax
from jax.experimental import pallas as pl
from jax.experimental.pallas import tpu as pltpu
from jax.experimental.pallas import tpu_sc as plsc

_B = 16384
_D = 128
_NC = 2
_NS = 16
_NW = _NC * _NS
_BPW = _B // _NW


def _body(h_hbm, dom_hbm, emb_hbm, out_hbm, idx_v, hbuf, sem_i, sem_h, sem_e,
          sem_o):
    wid = lax.axis_index("s") * _NC + lax.axis_index("c")
    base = wid * _BPW
    rows = pl.ds(base, _BPW)

    idx_cp = pltpu.make_async_copy(dom_hbm.at[rows], idx_v, sem_i)
    idx_cp.start()
    h_cp = pltpu.make_async_copy(h_hbm.at[rows], hbuf, sem_h)
    h_cp.start()
    idx_cp.wait()
    h_cp.wait()

    pltpu.async_copy(emb_hbm.at[idx_v], hbuf, sem_e, add=True).wait()

    out_cp = pltpu.make_async_copy(hbuf, out_hbm.at[rows], sem_o)
    out_cp.start()
    out_cp.wait()


@jax.jit
def _domain_token(h, domain, emb):
    mesh = plsc.VectorSubcoreMesh(core_axis_name="c", subcore_axis_name="s")
    return pl.kernel(
        _body,
        out_type=jax.ShapeDtypeStruct((_B, _D), jnp.float32),
        mesh=mesh,
        scratch_types=[
            pltpu.VMEM((_BPW,), jnp.int32),
            pltpu.VMEM((_BPW, _D), jnp.float32),
            pltpu.SemaphoreType.DMA,
            pltpu.SemaphoreType.DMA,
            pltpu.SemaphoreType.DMA,
            pltpu.SemaphoreType.DMA,
        ],
    )(h, domain, emb)


def kernel(h, domain, emb):
    return _domain_token(h, domain.astype(jnp.int32), emb)
